# baseline (speedup 1.0000x reference)
import functools

import jax
import jax.numpy as jnp
from jax import lax
from jax.experimental import pallas as pl
from jax.experimental.pallas import tpu as pltpu
from jax.experimental.pallas import tpu_sc as plsc

B, N, D, U = 128, 1024, 768, 1024
TILE_N = 512
NT = N // TILE_N


def _hvec_body(hid_ref, wh_ref, bias_ref, o_ref):
    o_ref[...] = (
        jnp.dot(hid_ref[...], wh_ref[...], preferred_element_type=jnp.float32)
        + bias_ref[...]
    )


def _hvec(hidden, Wh, bias2d):
    return pl.pallas_call(
        _hvec_body,
        out_shape=jax.ShapeDtypeStruct((B, U), jnp.float32),
    )(hidden, Wh, bias2d)


def _score_body(feat_ref, wf_ref, bf_ref, hvec_ref, ws_ref, bs_ref, alpha_ref, idx_ref):
    b = pl.program_id(0)
    n = pl.program_id(1)
    x = feat_ref[0]
    acc = jnp.dot(x, wf_ref[...], preferred_element_type=jnp.float32)
    t = jnp.tanh((acc + bf_ref[...]) + hvec_ref[0])
    tb = t.astype(jnp.bfloat16).astype(jnp.float32)
    logit = jnp.sum(tb * ws_ref[...], axis=1)
    alpha_ref[0, 0, pl.ds(n * TILE_N, TILE_N)] = logit

    @pl.when(n == NT - 1)
    def _():
        xrow = alpha_ref[0, 0, :].reshape(1, N) + bs_ref[0]
        m = jnp.max(xrow, axis=1, keepdims=True)
        e = jnp.exp(xrow - m)
        s = jnp.sum(e, axis=1, keepdims=True)
        a = e / s
        alpha_ref[0, 0, :] = a.reshape(N)
        am = jnp.max(a, axis=1, keepdims=True)
        col = lax.broadcasted_iota(jnp.int32, (1, N), 1)
        loc = jnp.min(jnp.where(a == am, col, N))
        idx_ref[0, 0, :] = jnp.full((128,), b * N, jnp.int32) + loc


def _scores(features, Wf, bf_row, hvec, ws_row_bf, bs):
    return pl.pallas_call(
        _score_body,
        grid=(B, NT),
        in_specs=[
            pl.BlockSpec((1, TILE_N, D), lambda b, n: (b, n, 0)),
            pl.BlockSpec((D, U), lambda b, n: (0, 0)),
            pl.BlockSpec((1, U), lambda b, n: (0, 0)),
            pl.BlockSpec((1, 1, U), lambda b, n: (b, 0, 0)),
            pl.BlockSpec((1, U), lambda b, n: (0, 0)),
            pl.BlockSpec(memory_space=pltpu.SMEM),
        ],
        out_specs=[
            pl.BlockSpec((1, 1, N), lambda b, n: (b, 0, 0)),
            pl.BlockSpec((1, 1, 128), lambda b, n: (b, 0, 0)),
        ],
        out_shape=[
            jax.ShapeDtypeStruct((B, 1, N), jnp.float32),
            jax.ShapeDtypeStruct((B, 1, 128), jnp.int32),
        ],
    )(features, Wf, bf_row, hvec, ws_row_bf, bs)


_NWU = 16
_RPW = B // _NWU


def _make_gather():
    info = plsc.get_sparse_core_info()
    nc = info.num_cores
    mesh = plsc.VectorSubcoreMesh(core_axis_name="c", subcore_axis_name="s")

    @functools.partial(
        pl.kernel,
        mesh=mesh,
        out_type=jax.ShapeDtypeStruct((B, D), jnp.float32),
        scratch_types=[
            pltpu.VMEM((_RPW,), jnp.int32),
            pltpu.VMEM((_RPW, D), jnp.float32),
            pltpu.SemaphoreType.DMA,
        ],
    )
    def gather(table_hbm, idx_hbm, out_hbm, idx_v, rows_v, sem):
        wid = lax.axis_index("s") * nc + lax.axis_index("c")

        @pl.when(wid < _NWU)
        def _():
            base = wid * _RPW
            pltpu.sync_copy(idx_hbm.at[pl.ds(base, _RPW)], idx_v)
            pltpu.async_copy(table_hbm.at[idx_v], rows_v, sem).wait()
            pltpu.sync_copy(rows_v, out_hbm.at[pl.ds(base, _RPW)])

    return gather


_gather = _make_gather()


def kernel(features, hidden, Wf, bf, Wh, bh, Ws, bs):
    hvec = _hvec(hidden, Wh, bh.reshape(1, U)).reshape(B, 1, U)
    ws_row_bf = Ws.reshape(1, U).astype(jnp.bfloat16).astype(jnp.float32)
    alpha3, idx3 = _scores(features, Wf, bf.reshape(1, U), hvec, ws_row_bf, bs)
    table = features.reshape(B * N, D)
    context = _gather(table, idx3[:, 0, 0])
    return (context, alpha3.reshape(B, N, 1))


# speedup vs baseline: 1.0485x; 1.0485x over previous
import functools

import jax
import jax.numpy as jnp
from jax import lax
from jax.experimental import pallas as pl
from jax.experimental.pallas import tpu as pltpu
from jax.experimental.pallas import tpu_sc as plsc

B, N, D, U = 128, 1024, 768, 1024
TB = 2
TM = TB * N
NTILES = B // TB


def _hvec_body(hid_ref, wh_ref, bias_ref, o_ref):
    o_ref[...] = (
        jnp.dot(hid_ref[...], wh_ref[...], preferred_element_type=jnp.float32)
        + bias_ref[...]
    )


def _hvec(hidden, Wh, bias2d):
    return pl.pallas_call(
        _hvec_body,
        out_shape=jax.ShapeDtypeStruct((B, U), jnp.float32),
    )(hidden, Wh, bias2d)


def _score_body(feat_ref, wf_ref, bf_ref, hvec_ref, ws_ref, bs_ref,
                alpha_ref, idx_ref, acc_ref):
    i = pl.program_id(0)

    @pl.when(i > 0)
    def _epilogue():
        prev = acc_ref[1 - (i % 2)]
        pre = (prev.reshape(TB, N, U) + bf_ref[...]) + hvec_ref[...]
        t = jnp.tanh(pre)
        lg = jnp.dot(t.reshape(TM, U), ws_ref[...],
                     preferred_element_type=jnp.float32)[:, 0]
        x = lg.reshape(TB, N) + bs_ref[0]
        m = jnp.max(x, axis=1, keepdims=True)
        e = jnp.exp(x - m)
        s = jnp.sum(e, axis=1, keepdims=True)
        a = e / s
        alpha_ref[...] = a.reshape(TB, 1, N)
        am = jnp.max(a, axis=1, keepdims=True)
        col = lax.broadcasted_iota(jnp.int32, (TB, N), 1)
        locs = jnp.min(jnp.where(a == am, col, N), axis=1)
        boff = (lax.broadcasted_iota(jnp.int32, (TB, 1, 128), 0)
                + (i - 1) * TB) * N
        idx_ref[...] = boff + locs.reshape(TB, 1, 1)

    @pl.when(i < NTILES)
    def _main_dot():
        acc_ref[i % 2] = jnp.dot(feat_ref[0], wf_ref[...],
                                 preferred_element_type=jnp.float32)


def _scores(features3, Wf, bf_row, hvec3, ws_pad, bs):
    return pl.pallas_call(
        _score_body,
        grid=(NTILES + 1,),
        in_specs=[
            pl.BlockSpec((1, TM, D), lambda i: (jnp.minimum(i, NTILES - 1), 0, 0)),
            pl.BlockSpec((D, U), lambda i: (0, 0)),
            pl.BlockSpec((1, U), lambda i: (0, 0)),
            pl.BlockSpec((TB, 1, U), lambda i: (jnp.maximum(i - 1, 0), 0, 0)),
            pl.BlockSpec((U, 128), lambda i: (0, 0)),
            pl.BlockSpec(memory_space=pltpu.SMEM),
        ],
        out_specs=[
            pl.BlockSpec((TB, 1, N), lambda i: (jnp.maximum(i - 1, 0), 0, 0)),
            pl.BlockSpec((TB, 1, 128), lambda i: (jnp.maximum(i - 1, 0), 0, 0)),
        ],
        out_shape=[
            jax.ShapeDtypeStruct((B, 1, N), jnp.float32),
            jax.ShapeDtypeStruct((B, 1, 128), jnp.int32),
        ],
        scratch_shapes=[pltpu.VMEM((2, TM, U), jnp.float32)],
    )(features3, Wf, bf_row, hvec3, ws_pad, bs)


_NWU = 16
_RPW = B // _NWU


def _make_gather():
    info = plsc.get_sparse_core_info()
    nc = info.num_cores
    mesh = plsc.VectorSubcoreMesh(core_axis_name="c", subcore_axis_name="s")

    @functools.partial(
        pl.kernel,
        mesh=mesh,
        out_type=jax.ShapeDtypeStruct((B, D), jnp.float32),
        scratch_types=[
            pltpu.VMEM((_RPW,), jnp.int32),
            pltpu.VMEM((_RPW, D), jnp.float32),
            pltpu.SemaphoreType.DMA,
        ],
    )
    def gather(table_hbm, idx_hbm, out_hbm, idx_v, rows_v, sem):
        wid = lax.axis_index("s") * nc + lax.axis_index("c")

        @pl.when(wid < _NWU)
        def _():
            base = wid * _RPW
            pltpu.sync_copy(idx_hbm.at[pl.ds(base, _RPW)], idx_v)
            pltpu.async_copy(table_hbm.at[idx_v], rows_v, sem).wait()
            pltpu.sync_copy(rows_v, out_hbm.at[pl.ds(base, _RPW)])

    return gather


_gather = _make_gather()


def kernel(features, hidden, Wf, bf, Wh, bh, Ws, bs):
    hvec = _hvec(hidden, Wh, bh.reshape(1, U)).reshape(B, 1, U)
    ws_pad = jnp.zeros((U, 128), jnp.float32).at[:, 0].set(Ws[:, 0])
    features3 = features.reshape(NTILES, TM, D)
    alpha3, idx3 = _scores(features3, Wf, bf.reshape(1, U), hvec, ws_pad, bs)
    table = features.reshape(B * N, D)
    context = _gather(table, idx3[:, 0, 0])
    return (context, alpha3.reshape(B, N, 1))
